# initial kernel scaffold (unmeasured)
import jax
import jax.numpy as jnp
from jax import lax
from jax.experimental import pallas as pl
from jax.experimental.pallas import tpu as pltpu

N_DEV = 32


def kernel(x, router_W, route_idx, expert_W, shared_W):
    n, d_model = x.shape
    e_local, _, h = expert_W.shape
    n_experts = router_W.shape[1]
    rows = n // N_DEV

    def body(x_ref, rw_ref, idx_ref, ew_ref, sw_ref, out_ref,
             partial_ref, comm_ref, send_sems, recv_sems):
        my_i = lax.axis_index("i")

        xv = x_ref[...]
        scores = jnp.dot(xv, rw_ref[...], preferred_element_type=jnp.float32)
        m = jnp.max(scores, axis=-1, keepdims=True)
        ex = jnp.exp(scores - m)
        probs = ex / jnp.sum(ex, axis=-1, keepdims=True)

        idx = idx_ref[...]
        je = lax.broadcasted_iota(jnp.int32, (n, n_experts), 1)
        routed = je == idx

        acc = jnp.zeros((n, h), dtype=jnp.float32)
        for k in range(e_local):
            e_id = my_i * e_local + k
            mask = routed & (je == e_id)
            coeff = jnp.sum(jnp.where(mask, probs, 0.0), axis=1)
            acc = acc + jnp.dot(
                xv * coeff[:, None], ew_ref[k],
                preferred_element_type=jnp.float32,
            )
        partial_ref[...] = acc

        comm_ref[my_i] = partial_ref[pl.ds(my_i * rows, rows), :]

        sends = []
        for step in range(1, N_DEV):
            dst = lax.rem(my_i + step, N_DEV)
            rdma = pltpu.make_async_remote_copy(
                src_ref=partial_ref.at[pl.ds(dst * rows, rows), :],
                dst_ref=comm_ref.at[my_i],
                send_sem=send_sems.at[step - 1],
                recv_sem=recv_sems.at[my_i],
                device_id=(dst,),
                device_id_type=pl.DeviceIdType.MESH,
            )
            rdma.start()
            sends.append(rdma)

        x_mine = x_ref[pl.ds(my_i * rows, rows), :]
        shared_mine = jnp.dot(x_mine, sw_ref[...],
                              preferred_element_type=jnp.float32)

        for step in range(1, N_DEV):
            src = lax.rem(my_i + step, N_DEV)
            recv = pltpu.make_async_remote_copy(
                src_ref=partial_ref.at[pl.ds(0, rows), :],
                dst_ref=comm_ref.at[src],
                send_sem=send_sems.at[step - 1],
                recv_sem=recv_sems.at[src],
                device_id=(src,),
                device_id_type=pl.DeviceIdType.MESH,
            )
            recv.wait_recv()

        out_ref[...] = shared_mine + jnp.sum(comm_ref[...], axis=0)

        for rdma in sends:
            rdma.wait_send()

    return pl.pallas_call(
        body,
        out_shape=jax.ShapeDtypeStruct((rows, h), jnp.float32),
        in_specs=[pl.BlockSpec(memory_space=pltpu.VMEM)] * 5,
        out_specs=pl.BlockSpec(memory_space=pltpu.VMEM),
        scratch_shapes=[
            pltpu.VMEM((n, h), jnp.float32),
            pltpu.VMEM((N_DEV, rows, h), jnp.float32),
            pltpu.SemaphoreType.DMA((N_DEV,)),
            pltpu.SemaphoreType.DMA((N_DEV,)),
        ],
        compiler_params=pltpu.CompilerParams(collective_id=0),
    )(x, router_W, route_idx, expert_W, shared_W)


# baseline (device time: 31614 ns/iter reference)
import jax
import jax.numpy as jnp
from jax import lax
from jax.experimental import pallas as pl
from jax.experimental.pallas import tpu as pltpu

N_DEV = 32


def kernel(x, router_W, route_idx, expert_W, shared_W):
    n, d_model = x.shape
    e_local, _, h = expert_W.shape
    n_experts = router_W.shape[1]
    rows = n // N_DEV

    def body(x_ref, rw_ref, idx_ref, ew_ref, sw_ref, out_ref,
             partial_ref, comm_ref, send_sems, recv_sems):
        my_i = lax.axis_index("i")

        xv = x_ref[...]
        scores = jnp.dot(xv, rw_ref[...], preferred_element_type=jnp.float32)
        m = jnp.max(scores, axis=-1, keepdims=True)
        ex = jnp.exp(scores - m)
        probs = ex / jnp.sum(ex, axis=-1, keepdims=True)

        idx = idx_ref[...]
        je = lax.broadcasted_iota(jnp.int32, (n, n_experts), 1)
        routed = je == idx

        acc = jnp.zeros((n, h), dtype=jnp.float32)
        for k in range(e_local):
            e_id = my_i * e_local + k
            mask = routed & (je == e_id)
            coeff = jnp.sum(jnp.where(mask, probs, 0.0), axis=1)
            acc = acc + jnp.dot(
                xv * coeff[:, None], ew_ref[k],
                preferred_element_type=jnp.float32,
            )
        partial_ref[...] = acc

        comm_ref[my_i] = partial_ref[pl.ds(my_i * rows, rows), :]

        sends = []
        for step in range(1, N_DEV):
            dst = lax.rem(my_i + step, N_DEV)
            rdma = pltpu.make_async_remote_copy(
                src_ref=partial_ref.at[pl.ds(dst * rows, rows), :],
                dst_ref=comm_ref.at[my_i],
                send_sem=send_sems.at[step - 1],
                recv_sem=recv_sems.at[my_i],
                device_id=(dst,),
                device_id_type=pl.DeviceIdType.MESH,
            )
            rdma.start()
            sends.append(rdma)

        x_mine = x_ref[pl.ds(my_i * rows, rows), :]
        shared_mine = jnp.dot(x_mine, sw_ref[...],
                              preferred_element_type=jnp.float32)

        for step in range(1, N_DEV):
            src = lax.rem(my_i + step, N_DEV)
            recv = pltpu.make_async_remote_copy(
                src_ref=partial_ref.at[pl.ds(0, rows), :],
                dst_ref=comm_ref.at[src],
                send_sem=send_sems.at[step - 1],
                recv_sem=recv_sems.at[src],
                device_id=(src,),
                device_id_type=pl.DeviceIdType.MESH,
            )
            recv.wait_recv()

        out_ref[...] = shared_mine + jnp.sum(comm_ref[...], axis=0)

        for rdma in sends:
            rdma.wait_send()

    return pl.pallas_call(
        body,
        out_shape=jax.ShapeDtypeStruct((rows, h), jnp.float32),
        in_specs=[pl.BlockSpec(memory_space=pltpu.VMEM)] * 5,
        out_specs=pl.BlockSpec(memory_space=pltpu.VMEM),
        scratch_shapes=[
            pltpu.VMEM((n, h), jnp.float32),
            pltpu.VMEM((N_DEV, rows, h), jnp.float32),
            pltpu.SemaphoreType.DMA((N_DEV,)),
            pltpu.SemaphoreType.DMA((N_DEV,)),
        ],
    )(x, router_W, route_idx, expert_W, shared_W)


# device time: 26348 ns/iter; 1.1999x vs baseline; 1.1999x over previous
import jax
import jax.numpy as jnp
from jax import lax
from jax.experimental import pallas as pl
from jax.experimental.pallas import tpu as pltpu

N_DEV = 32


def kernel(x, router_W, route_idx, expert_W, shared_W):
    n, d_model = x.shape
    e_local, _, h = expert_W.shape
    n_experts = router_W.shape[1]
    rows = n // N_DEV

    def body(x_ref, rw_ref, idx_ref, ew_ref, sw_ref, out_ref,
             partial_ref, comm_ref, send_sems, recv_sems):
        my_i = lax.axis_index("i")

        xv = x_ref[...]
        scores = jnp.dot(xv, rw_ref[...], preferred_element_type=jnp.float32)
        m = jnp.max(scores, axis=-1, keepdims=True)
        ex = jnp.exp(scores - m)
        probs = ex / jnp.sum(ex, axis=-1, keepdims=True)

        idx = idx_ref[...]
        je = lax.broadcasted_iota(jnp.int32, (n, n_experts), 1)
        routed = je == idx

        acc = jnp.zeros((n, h), dtype=jnp.float32)
        for k in range(e_local):
            e_id = my_i * e_local + k
            mask = routed & (je == e_id)
            coeff = jnp.sum(jnp.where(mask, probs, 0.0), axis=1)
            acc = acc + jnp.dot(
                (xv * coeff[:, None]).astype(jnp.bfloat16),
                ew_ref[k][...].astype(jnp.bfloat16),
                preferred_element_type=jnp.float32,
            )
        partial_ref[...] = acc.astype(jnp.bfloat16)

        comm_ref[my_i] = partial_ref[pl.ds(my_i * rows, rows), :]

        sends = []
        for step in range(1, N_DEV):
            dst = lax.rem(my_i + step, N_DEV)
            rdma = pltpu.make_async_remote_copy(
                src_ref=partial_ref.at[pl.ds(dst * rows, rows), :],
                dst_ref=comm_ref.at[my_i],
                send_sem=send_sems.at[step - 1],
                recv_sem=recv_sems.at[my_i],
                device_id=(dst,),
                device_id_type=pl.DeviceIdType.MESH,
            )
            rdma.start()
            sends.append(rdma)

        x_mine = x_ref[pl.ds(my_i * rows, rows), :]
        shared_mine = jnp.dot(x_mine, sw_ref[...],
                              preferred_element_type=jnp.float32)

        for step in range(1, N_DEV):
            src = lax.rem(my_i + step, N_DEV)
            recv = pltpu.make_async_remote_copy(
                src_ref=partial_ref.at[pl.ds(0, rows), :],
                dst_ref=comm_ref.at[src],
                send_sem=send_sems.at[step - 1],
                recv_sem=recv_sems.at[src],
                device_id=(src,),
                device_id_type=pl.DeviceIdType.MESH,
            )
            recv.wait_recv()

        out_ref[...] = shared_mine + jnp.sum(
            comm_ref[...].astype(jnp.float32), axis=0)

        for rdma in sends:
            rdma.wait_send()

    return pl.pallas_call(
        body,
        out_shape=jax.ShapeDtypeStruct((rows, h), jnp.float32),
        in_specs=[pl.BlockSpec(memory_space=pltpu.VMEM)] * 5,
        out_specs=pl.BlockSpec(memory_space=pltpu.VMEM),
        scratch_shapes=[
            pltpu.VMEM((n, h), jnp.bfloat16),
            pltpu.VMEM((N_DEV, rows, h), jnp.bfloat16),
            pltpu.SemaphoreType.DMA((N_DEV,)),
            pltpu.SemaphoreType.DMA((N_DEV,)),
        ],
    )(x, router_W, route_idx, expert_W, shared_W)


# device time: 17650 ns/iter; 1.7912x vs baseline; 1.4928x over previous
import jax
import jax.numpy as jnp
from jax import lax
from jax.experimental import pallas as pl
from jax.experimental.pallas import tpu as pltpu

N_DEV = 32


def kernel(x, router_W, route_idx, expert_W, shared_W):
    n, d_model = x.shape
    e_local, _, h = expert_W.shape
    n_experts = router_W.shape[1]
    rows = n // N_DEV

    def body(x_ref, rw_ref, idx_ref, ew_ref, sw_ref, out_ref,
             partial_ref, comm_ref, send_sems, recv_sems):
        my_i = lax.axis_index("i")

        barrier_sem = pltpu.get_barrier_semaphore()
        for step in range(1, N_DEV):
            peer = lax.rem(my_i + step, N_DEV)
            pl.semaphore_signal(
                barrier_sem, inc=1,
                device_id=(peer,), device_id_type=pl.DeviceIdType.MESH,
            )

        xv = x_ref[...]
        scores = jnp.dot(xv, rw_ref[...], preferred_element_type=jnp.float32)
        m = jnp.max(scores, axis=-1, keepdims=True)
        ex = jnp.exp(scores - m)
        probs = ex / jnp.sum(ex, axis=-1, keepdims=True)

        idx = idx_ref[...]
        je = lax.broadcasted_iota(jnp.int32, (n, n_experts), 1)
        routed = je == idx

        scaled = []
        for k in range(e_local):
            e_id = my_i * e_local + k
            mask = routed & (je == e_id)
            coeff = jnp.sum(jnp.where(mask, probs, 0.0), axis=1)
            scaled.append((xv * coeff[:, None]).astype(jnp.bfloat16))
        xcat = jnp.concatenate(scaled, axis=1)
        xrot = pltpu.roll(xcat, -my_i * rows, 0)
        wcat = ew_ref[...].astype(jnp.bfloat16).reshape(e_local * d_model, h)

        n_chunks = 4
        crows = n // n_chunks
        d_per_chunk = N_DEV // n_chunks
        sends = []
        for c in range(n_chunks):
            a = jnp.dot(xrot[c * crows:(c + 1) * crows, :], wcat,
                        preferred_element_type=jnp.float32)
            partial_ref[c * crows:(c + 1) * crows, :] = a.astype(jnp.bfloat16)
            if c == 0:
                pl.semaphore_wait(barrier_sem, N_DEV - 1)
            for s in range(c * d_per_chunk, (c + 1) * d_per_chunk):
                if s == 0:
                    continue
                dst = lax.rem(my_i + s, N_DEV)
                rdma = pltpu.make_async_remote_copy(
                    src_ref=partial_ref.at[pl.ds(s * rows, rows), :],
                    dst_ref=comm_ref.at[my_i],
                    send_sem=send_sems.at[s - 1],
                    recv_sem=recv_sems.at[my_i],
                    device_id=(dst,),
                    device_id_type=pl.DeviceIdType.MESH,
                )
                rdma.start()
                sends.append(rdma)

        x_mine = x_ref[pl.ds(my_i * rows, rows), :]
        shared_mine = jnp.dot(x_mine, sw_ref[...],
                              preferred_element_type=jnp.float32)
        out_acc = shared_mine + partial_ref[0:rows, :].astype(jnp.float32)

        for step in range(1, N_DEV):
            src = lax.rem(my_i - step + N_DEV, N_DEV)
            recv = pltpu.make_async_remote_copy(
                src_ref=partial_ref.at[pl.ds(0, rows), :],
                dst_ref=comm_ref.at[src],
                send_sem=send_sems.at[step - 1],
                recv_sem=recv_sems.at[src],
                device_id=(src,),
                device_id_type=pl.DeviceIdType.MESH,
            )
            recv.wait_recv()
            out_acc = out_acc + comm_ref[src].astype(jnp.float32)

        out_ref[...] = out_acc

        for rdma in sends:
            rdma.wait_send()

    return pl.pallas_call(
        body,
        out_shape=jax.ShapeDtypeStruct((rows, h), jnp.float32),
        in_specs=[pl.BlockSpec(memory_space=pltpu.VMEM)] * 5,
        out_specs=pl.BlockSpec(memory_space=pltpu.VMEM),
        scratch_shapes=[
            pltpu.VMEM((n, h), jnp.bfloat16),
            pltpu.VMEM((N_DEV, rows, h), jnp.bfloat16),
            pltpu.SemaphoreType.DMA((N_DEV,)),
            pltpu.SemaphoreType.DMA((N_DEV,)),
        ],
        compiler_params=pltpu.CompilerParams(collective_id=0),
    )(x, router_W, route_idx, expert_W, shared_W)
